# baseline (speedup 1.0000x reference)
import functools

import jax
import jax.numpy as jnp
from jax import lax
from jax.experimental import pallas as pl
from jax.experimental.pallas import tpu as pltpu
from jax.experimental.pallas import tpu_sc as plsc

_DEG = 8
_CH = 16
_VB = 512


def _bcast_lane(vec, lane):
    idx = jnp.full((16, 1), lane, dtype=jnp.int32)
    dn = lax.GatherDimensionNumbers(
        offset_dims=(), collapsed_slice_dims=(0,), start_index_map=(0,))
    return lax.gather(vec, idx, dn, (1,),
                      mode=lax.GatherScatterMode.PROMISE_IN_BOUNDS)


def _make_spmv(V, C):
    info = plsc.get_sparse_core_info()
    nw = info.num_cores * info.num_subcores
    rpw = V // nw
    nch = rpw // _CH
    ech = _CH * _DEG
    nj = C // 16
    if C <= 64:
        nbuf = 4
    elif C <= 128:
        nbuf = 3
    else:
        nbuf = 2
    while nch % nbuf:
        nbuf -= 1
    mesh = plsc.VectorSubcoreMesh(core_axis_name="c", subcore_axis_name="s")

    @functools.partial(
        pl.kernel, mesh=mesh,
        out_type=jax.ShapeDtypeStruct((V, C), jnp.float32),
        scratch_types=[
            pltpu.VMEM((nbuf, ech), jnp.int32),
            pltpu.VMEM((nbuf, ech), jnp.float32),
            pltpu.VMEM((nbuf, ech, C), jnp.float32),
            pltpu.VMEM((nbuf, _CH, C), jnp.float32),
        ] + [pltpu.SemaphoreType.DMA] * (4 * nbuf),
        compiler_params=pltpu.CompilerParams(use_tc_tiling_on_sc=False),
    )
    def spmv(xt_hbm, cols_hbm, vals_hbm, out_hbm, colbuf, valbuf, gbuf, accbuf, *sems):
        csem, vsem = sems[0:nbuf], sems[nbuf:2 * nbuf]
        gsem, osem = sems[2 * nbuf:3 * nbuf], sems[3 * nbuf:4 * nbuf]
        wid = lax.axis_index("s") * info.num_cores + lax.axis_index("c")
        row_base = wid * rpw

        def e_sl(ch):
            return pl.ds((row_base + ch * _CH) * _DEG, ech)

        def out_sl(ch):
            return pl.ds(row_base + ch * _CH, _CH)

        def compute(p):
            for t in range(_CH // 2):
                vv = valbuf[p, pl.ds(16 * t, 16)]
                acc0 = [jnp.zeros((16,), jnp.float32)] * nj
                acc1 = [jnp.zeros((16,), jnp.float32)] * nj
                for d in range(_DEG):
                    w0 = _bcast_lane(vv, d)
                    w1 = _bcast_lane(vv, _DEG + d)
                    for j in range(nj):
                        acc0[j] = acc0[j] + w0 * gbuf[p, 16 * t + d, pl.ds(16 * j, 16)]
                        acc1[j] = acc1[j] + w1 * gbuf[p, 16 * t + _DEG + d, pl.ds(16 * j, 16)]
                for j in range(nj):
                    accbuf[p, 2 * t, pl.ds(16 * j, 16)] = acc0[j]
                    accbuf[p, 2 * t + 1, pl.ds(16 * j, 16)] = acc1[j]

        for k in range(nbuf):
            pltpu.async_copy(cols_hbm.at[e_sl(k)], colbuf.at[k], csem[k])
            pltpu.async_copy(vals_hbm.at[e_sl(k)], valbuf.at[k], vsem[k])
        for k in range(nbuf - 1):
            pltpu.make_async_copy(cols_hbm.at[e_sl(k)], colbuf.at[k], csem[k]).wait()
            pltpu.async_copy(xt_hbm.at[colbuf.at[k]], gbuf.at[k], gsem[k])

        def body(chb, carry):
            for b in range(nbuf):
                ch = chb * nbuf + b
                p = b
                q = (b - 1) % nbuf
                pltpu.make_async_copy(
                    xt_hbm.at[colbuf.at[p]], gbuf.at[p], gsem[p]).wait()

                @pl.when(ch + nbuf - 1 < nch)
                def _():
                    pltpu.make_async_copy(
                        cols_hbm.at[e_sl(ch + nbuf - 1)], colbuf.at[q], csem[q]).wait()
                    pltpu.async_copy(xt_hbm.at[colbuf.at[q]], gbuf.at[q], gsem[q])

                pltpu.make_async_copy(
                    vals_hbm.at[e_sl(ch)], valbuf.at[p], vsem[p]).wait()

                @pl.when(ch >= nbuf)
                def _():
                    pltpu.make_async_copy(
                        accbuf.at[p], out_hbm.at[out_sl(ch - nbuf)], osem[p]).wait()

                compute(p)
                pltpu.async_copy(accbuf.at[p], out_hbm.at[out_sl(ch)], osem[p])

                @pl.when(ch + nbuf < nch)
                def _():
                    pltpu.async_copy(cols_hbm.at[e_sl(ch + nbuf)], colbuf.at[p], csem[p])
                    pltpu.async_copy(vals_hbm.at[e_sl(ch + nbuf)], valbuf.at[p], vsem[p])
            return carry

        lax.fori_loop(0, nch // nbuf, body, 0)
        for k in range(nbuf):
            pltpu.make_async_copy(
                accbuf.at[k], out_hbm.at[out_sl(nch - nbuf + k)], osem[k]).wait()

    return spmv


def _spmv(xt, cols, vals, V, C):
    return _make_spmv(V, C)(xt, cols, vals)


def _mm3(x0, x1, m, wa, wb, wc, V, C, C2):
    ng = V // _VB

    def body(x0_ref, x1_ref, m_ref, a_ref, b_ref, c_ref, y_ref, st_ref):
        dot = functools.partial(jnp.dot, preferred_element_type=jnp.float32)
        x2 = 2.0 * m_ref[...] - x0_ref[...]
        y = dot(x0_ref[...], a_ref[...])
        y = y + dot(x1_ref[...], b_ref[...])
        y = y + dot(x2, c_ref[...])
        y_ref[...] = y

        @pl.when(pl.program_id(0) == 0)
        def _():
            st_ref[...] = jnp.zeros_like(st_ref)

        st_ref[0:1, :] += jnp.sum(y, axis=0, keepdims=True)
        st_ref[1:2, :] += jnp.sum(y * y, axis=0, keepdims=True)

    xspec = pl.BlockSpec((_VB, C), lambda i: (i, 0))
    wspec = pl.BlockSpec((C, C2), lambda i: (0, 0))
    return pl.pallas_call(
        body,
        grid=(ng,),
        in_specs=[xspec, xspec, xspec, wspec, wspec, wspec],
        out_specs=[pl.BlockSpec((_VB, C2), lambda i: (i, 0)),
                   pl.BlockSpec((8, C2), lambda i: (0, 0))],
        out_shape=[jax.ShapeDtypeStruct((V, C2), jnp.float32),
                   jax.ShapeDtypeStruct((8, C2), jnp.float32)],
    )(x0, x1, m, wa, wb, wc)


def _bn(y, st, gamma, beta, V, C2, want_out, want_pool):
    F = C2 // 2
    ng = V // _VB
    n = 2.0 * V

    def body(y_ref, st_ref, g_ref, b_ref, *out_refs):
        s = st_ref[0:1, :]
        q = st_ref[1:2, :]
        mean = (s[:, :F] + s[:, F:]) / n
        var = (q[:, :F] + q[:, F:]) / n - mean * mean
        scale = g_ref[...] / jnp.sqrt(var + 1e-5)
        shift = b_ref[...] - mean * scale
        yb = y_ref[...]
        z0 = jnp.maximum(yb[:, :F] * scale + shift, 0.0)
        z1 = jnp.maximum(yb[:, F:] * scale + shift, 0.0)
        k = 0
        if want_out:
            out_refs[k][0, :, :] = z0
            out_refs[k][1, :, :] = z1
            k += 1
        if want_pool:
            p0 = jnp.max(z0.reshape(_VB // 4, 4, F), axis=1)
            p1 = jnp.max(z1.reshape(_VB // 4, 4, F), axis=1)
            out_refs[k][:, :F] = p0
            out_refs[k][:, F:] = p1
            k += 1
        if not (want_out or want_pool):
            out_refs[0][:, :F] = z0
            out_refs[0][:, F:] = z1

    out_specs, out_shape = [], []
    if want_out:
        out_specs.append(pl.BlockSpec((2, _VB, F), lambda i: (0, i, 0)))
        out_shape.append(jax.ShapeDtypeStruct((2, V, F), jnp.float32))
    if want_pool:
        out_specs.append(pl.BlockSpec((_VB // 4, C2), lambda i: (i, 0)))
        out_shape.append(jax.ShapeDtypeStruct((V // 4, C2), jnp.float32))
    if not (want_out or want_pool):
        out_specs.append(pl.BlockSpec((_VB, C2), lambda i: (i, 0)))
        out_shape.append(jax.ShapeDtypeStruct((V, C2), jnp.float32))

    res = pl.pallas_call(
        body,
        grid=(ng,),
        in_specs=[pl.BlockSpec((_VB, C2), lambda i: (i, 0)),
                  pl.BlockSpec((8, C2), lambda i: (0, 0)),
                  pl.BlockSpec((1, F), lambda i: (0, 0)),
                  pl.BlockSpec((1, F), lambda i: (0, 0))],
        out_specs=out_specs,
        out_shape=out_shape,
    )(y, st, gamma.reshape(1, F), beta.reshape(1, F))
    return res if len(out_shape) > 1 else res[0]


def _blkdiag2(w):
    fi, fo = w.shape
    z = jnp.zeros((2 * fi, 2 * fo), jnp.float32)
    return z.at[:fi, :fo].set(w).at[fi:, fo:].set(w)


def _conv_bn(xt, cols, vals, W, gamma, beta, V, fin, fout, want_out, want_pool):
    C = 2 * fin
    C2 = 2 * fout
    w0, w1, w2 = W[0::3], W[1::3], W[2::3]
    x1 = _spmv(xt, cols, vals, V, C)
    m = _spmv(x1, cols, vals, V, C)
    y, st = _mm3(xt, x1, m, _blkdiag2(w0), _blkdiag2(w1), _blkdiag2(w2), V, C, C2)
    return _bn(y, st, gamma, beta, V, C2, want_out, want_pool)


def kernel(x, rows0, cols0, vals0, rows1, cols1, vals1, rows2, cols2, vals2,
           W1a, g1a, b1a, W1b, g1b, b1b, W2, g2, b2, W3, g3, b3):
    B, V0, F0 = x.shape
    V1, V2 = V0 // 4, V0 // 16
    xt0 = jnp.transpose(x, (1, 0, 2)).reshape(V0, B * F0)
    h = _conv_bn(xt0, cols0, vals0, W1a, g1a, b1a, V0, 16, 32, False, False)
    out1, p1 = _conv_bn(h, cols0, vals0, W1b, g1b, b1b, V0, 32, 64, True, True)
    out2, p2 = _conv_bn(p1, cols1, vals1, W2, g2, b2, V1, 64, 128, True, True)
    out3 = _conv_bn(p2, cols2, vals2, W3, g3, b3, V2, 128, 256, True, False)
    return (out3, out2, out1)


# speedup vs baseline: 1.1081x; 1.1081x over previous
import functools

import jax
import jax.numpy as jnp
from jax import lax
from jax.experimental import pallas as pl
from jax.experimental.pallas import tpu as pltpu
from jax.experimental.pallas import tpu_sc as plsc

_DEG = 8
_CH = 16
_VB = 512


def _bcast_lane(vec, lane):
    idx = jnp.full((16, 1), lane, dtype=jnp.int32)
    dn = lax.GatherDimensionNumbers(
        offset_dims=(), collapsed_slice_dims=(0,), start_index_map=(0,))
    return lax.gather(vec, idx, dn, (1,),
                      mode=lax.GatherScatterMode.PROMISE_IN_BOUNDS)


def _make_spmv(V, C):
    info = plsc.get_sparse_core_info()
    nw = info.num_cores * info.num_subcores
    rpw = V // nw
    nch = rpw // _CH
    ech = _CH * _DEG
    nj = C // 16
    nbuf = 2
    mesh = plsc.VectorSubcoreMesh(core_axis_name="c", subcore_axis_name="s")

    @functools.partial(
        pl.kernel, mesh=mesh,
        out_type=jax.ShapeDtypeStruct((V, C), jnp.float32),
        scratch_types=[
            pltpu.VMEM((nbuf, ech), jnp.int32),
            pltpu.VMEM((nbuf, ech), jnp.float32),
            pltpu.VMEM((nbuf, ech, C), jnp.float32),
            pltpu.VMEM((nbuf, _CH, C), jnp.float32),
        ] + [pltpu.SemaphoreType.DMA] * (4 * nbuf),
        compiler_params=pltpu.CompilerParams(use_tc_tiling_on_sc=False),
    )
    def spmv(xt_hbm, cols_hbm, vals_hbm, out_hbm, colbuf, valbuf, gbuf, accbuf, *sems):
        csem, vsem = sems[0:nbuf], sems[nbuf:2 * nbuf]
        gsem, osem = sems[2 * nbuf:3 * nbuf], sems[3 * nbuf:4 * nbuf]
        wid = lax.axis_index("s") * info.num_cores + lax.axis_index("c")
        row_base = wid * rpw

        def e_sl(ch):
            return pl.ds((row_base + ch * _CH) * _DEG, ech)

        def out_sl(ch):
            return pl.ds(row_base + ch * _CH, _CH)

        def compute(p):
            for t in range(_CH // 2):
                vv = valbuf[p, pl.ds(16 * t, 16)]
                acc0 = [jnp.zeros((16,), jnp.float32)] * nj
                acc1 = [jnp.zeros((16,), jnp.float32)] * nj
                for d in range(_DEG):
                    w0 = _bcast_lane(vv, d)
                    w1 = _bcast_lane(vv, _DEG + d)
                    for j in range(nj):
                        acc0[j] = acc0[j] + w0 * gbuf[p, 16 * t + d, pl.ds(16 * j, 16)]
                        acc1[j] = acc1[j] + w1 * gbuf[p, 16 * t + _DEG + d, pl.ds(16 * j, 16)]
                for j in range(nj):
                    accbuf[p, 2 * t, pl.ds(16 * j, 16)] = acc0[j]
                    accbuf[p, 2 * t + 1, pl.ds(16 * j, 16)] = acc1[j]

        for k in range(nbuf):
            pltpu.async_copy(cols_hbm.at[e_sl(k)], colbuf.at[k], csem[k])
            pltpu.async_copy(vals_hbm.at[e_sl(k)], valbuf.at[k], vsem[k])
        for k in range(nbuf - 1):
            pltpu.make_async_copy(cols_hbm.at[e_sl(k)], colbuf.at[k], csem[k]).wait()
            pltpu.async_copy(xt_hbm.at[colbuf.at[k]], gbuf.at[k], gsem[k])

        def body(chb, carry):
            for b in range(nbuf):
                ch = chb * nbuf + b
                p = b
                q = (b - 1) % nbuf
                pltpu.make_async_copy(
                    xt_hbm.at[colbuf.at[p]], gbuf.at[p], gsem[p]).wait()

                @pl.when(ch + nbuf - 1 < nch)
                def _():
                    pltpu.make_async_copy(
                        cols_hbm.at[e_sl(ch + nbuf - 1)], colbuf.at[q], csem[q]).wait()
                    pltpu.async_copy(xt_hbm.at[colbuf.at[q]], gbuf.at[q], gsem[q])

                pltpu.make_async_copy(
                    vals_hbm.at[e_sl(ch)], valbuf.at[p], vsem[p]).wait()

                @pl.when(ch >= nbuf)
                def _():
                    pltpu.make_async_copy(
                        accbuf.at[p], out_hbm.at[out_sl(ch - nbuf)], osem[p]).wait()

                compute(p)
                pltpu.async_copy(accbuf.at[p], out_hbm.at[out_sl(ch)], osem[p])

                @pl.when(ch + nbuf < nch)
                def _():
                    pltpu.async_copy(cols_hbm.at[e_sl(ch + nbuf)], colbuf.at[p], csem[p])
                    pltpu.async_copy(vals_hbm.at[e_sl(ch + nbuf)], valbuf.at[p], vsem[p])
            return carry

        lax.fori_loop(0, nch // nbuf, body, 0)
        for k in range(nbuf):
            pltpu.make_async_copy(
                accbuf.at[k], out_hbm.at[out_sl(nch - nbuf + k)], osem[k]).wait()

    return spmv


def _spmv(xt, cols, vals, V, C):
    return _make_spmv(V, C)(xt, cols, vals)


def _mm3(x0, x1, m, wa, wb, wc, V, C, C2):
    ng = V // _VB

    def body(x0_ref, x1_ref, m_ref, a_ref, b_ref, c_ref, y_ref, st_ref):
        dot = functools.partial(jnp.dot, preferred_element_type=jnp.float32)
        x2 = 2.0 * m_ref[...] - x0_ref[...]
        y = dot(x0_ref[...], a_ref[...])
        y = y + dot(x1_ref[...], b_ref[...])
        y = y + dot(x2, c_ref[...])
        y_ref[...] = y

        @pl.when(pl.program_id(0) == 0)
        def _():
            st_ref[...] = jnp.zeros_like(st_ref)

        st_ref[0:1, :] += jnp.sum(y, axis=0, keepdims=True)
        st_ref[1:2, :] += jnp.sum(y * y, axis=0, keepdims=True)

    xspec = pl.BlockSpec((_VB, C), lambda i: (i, 0))
    wspec = pl.BlockSpec((C, C2), lambda i: (0, 0))
    return pl.pallas_call(
        body,
        grid=(ng,),
        in_specs=[xspec, xspec, xspec, wspec, wspec, wspec],
        out_specs=[pl.BlockSpec((_VB, C2), lambda i: (i, 0)),
                   pl.BlockSpec((8, C2), lambda i: (0, 0))],
        out_shape=[jax.ShapeDtypeStruct((V, C2), jnp.float32),
                   jax.ShapeDtypeStruct((8, C2), jnp.float32)],
    )(x0, x1, m, wa, wb, wc)


def _bn(y, st, gamma, beta, V, C2, want_out, want_pool):
    F = C2 // 2
    ng = V // _VB
    n = 2.0 * V

    def body(y_ref, st_ref, g_ref, b_ref, *out_refs):
        s = st_ref[0:1, :]
        q = st_ref[1:2, :]
        mean = (s[:, :F] + s[:, F:]) / n
        var = (q[:, :F] + q[:, F:]) / n - mean * mean
        scale = g_ref[...] / jnp.sqrt(var + 1e-5)
        shift = b_ref[...] - mean * scale
        yb = y_ref[...]
        z0 = jnp.maximum(yb[:, :F] * scale + shift, 0.0)
        z1 = jnp.maximum(yb[:, F:] * scale + shift, 0.0)
        k = 0
        if want_out:
            out_refs[k][0, :, :] = z0
            out_refs[k][1, :, :] = z1
            k += 1
        if want_pool:
            p0 = jnp.max(z0.reshape(_VB // 4, 4, F), axis=1)
            p1 = jnp.max(z1.reshape(_VB // 4, 4, F), axis=1)
            out_refs[k][:, :F] = p0
            out_refs[k][:, F:] = p1
            k += 1
        if not (want_out or want_pool):
            out_refs[0][:, :F] = z0
            out_refs[0][:, F:] = z1

    out_specs, out_shape = [], []
    if want_out:
        out_specs.append(pl.BlockSpec((2, _VB, F), lambda i: (0, i, 0)))
        out_shape.append(jax.ShapeDtypeStruct((2, V, F), jnp.float32))
    if want_pool:
        out_specs.append(pl.BlockSpec((_VB // 4, C2), lambda i: (i, 0)))
        out_shape.append(jax.ShapeDtypeStruct((V // 4, C2), jnp.float32))
    if not (want_out or want_pool):
        out_specs.append(pl.BlockSpec((_VB, C2), lambda i: (i, 0)))
        out_shape.append(jax.ShapeDtypeStruct((V, C2), jnp.float32))

    res = pl.pallas_call(
        body,
        grid=(ng,),
        in_specs=[pl.BlockSpec((_VB, C2), lambda i: (i, 0)),
                  pl.BlockSpec((8, C2), lambda i: (0, 0)),
                  pl.BlockSpec((1, F), lambda i: (0, 0)),
                  pl.BlockSpec((1, F), lambda i: (0, 0))],
        out_specs=out_specs,
        out_shape=out_shape,
    )(y, st, gamma.reshape(1, F), beta.reshape(1, F))
    return res if len(out_shape) > 1 else res[0]


def _blkdiag2(w):
    fi, fo = w.shape
    z = jnp.zeros((2 * fi, 2 * fo), jnp.float32)
    return z.at[:fi, :fo].set(w).at[fi:, fo:].set(w)


def _conv_bn(xt, cols, vals, W, gamma, beta, V, fin, fout, want_out, want_pool):
    C = 2 * fin
    C2 = 2 * fout
    w0, w1, w2 = W[0::3], W[1::3], W[2::3]
    x1 = _spmv(xt, cols, vals, V, C)
    m = _spmv(x1, cols, vals, V, C)
    y, st = _mm3(xt, x1, m, _blkdiag2(w0), _blkdiag2(w1), _blkdiag2(w2), V, C, C2)
    return _bn(y, st, gamma, beta, V, C2, want_out, want_pool)


def kernel(x, rows0, cols0, vals0, rows1, cols1, vals1, rows2, cols2, vals2,
           W1a, g1a, b1a, W1b, g1b, b1b, W2, g2, b2, W3, g3, b3):
    B, V0, F0 = x.shape
    V1, V2 = V0 // 4, V0 // 16
    xt0 = jnp.transpose(x, (1, 0, 2)).reshape(V0, B * F0)
    h = _conv_bn(xt0, cols0, vals0, W1a, g1a, b1a, V0, 16, 32, False, False)
    out1, p1 = _conv_bn(h, cols0, vals0, W1b, g1b, b1b, V0, 32, 64, True, True)
    out2, p2 = _conv_bn(p1, cols1, vals1, W2, g2, b2, V1, 64, 128, True, True)
    out3 = _conv_bn(p2, cols2, vals2, W3, g3, b3, V2, 128, 256, True, False)
    return (out3, out2, out1)


# speedup vs baseline: 1.1565x; 1.0436x over previous
import functools

import jax
import jax.numpy as jnp
from jax import lax
from jax.experimental import pallas as pl
from jax.experimental.pallas import tpu as pltpu
from jax.experimental.pallas import tpu_sc as plsc

_DEG = 8
_CH = 16
_VB = 512


def _bcast_lane(vec, lane):
    idx = jnp.full((16, 1), lane, dtype=jnp.int32)
    dn = lax.GatherDimensionNumbers(
        offset_dims=(), collapsed_slice_dims=(0,), start_index_map=(0,))
    return lax.gather(vec, idx, dn, (1,),
                      mode=lax.GatherScatterMode.PROMISE_IN_BOUNDS)


def _make_spmv(V, C):
    info = plsc.get_sparse_core_info()
    nw = info.num_cores * info.num_subcores
    rpw = V // nw
    nch = rpw // _CH
    ech = _CH * _DEG
    nj = C // 16
    nbuf = 2
    mesh = plsc.VectorSubcoreMesh(core_axis_name="c", subcore_axis_name="s")

    @functools.partial(
        pl.kernel, mesh=mesh,
        out_type=jax.ShapeDtypeStruct((V, C), jnp.float32),
        scratch_types=[
            pltpu.VMEM((nbuf, ech), jnp.int32),
            pltpu.VMEM((nbuf, ech), jnp.float32),
            pltpu.VMEM((nbuf, ech, C), jnp.float32),
            pltpu.VMEM((nbuf, _CH, C), jnp.float32),
        ] + [pltpu.SemaphoreType.DMA] * (4 * nbuf),
        compiler_params=pltpu.CompilerParams(use_tc_tiling_on_sc=False),
    )
    def spmv(xt_hbm, cols_hbm, vals_hbm, out_hbm, colbuf, valbuf, gbuf, accbuf, *sems):
        csem, vsem = sems[0:nbuf], sems[nbuf:2 * nbuf]
        gsem, osem = sems[2 * nbuf:3 * nbuf], sems[3 * nbuf:4 * nbuf]
        wid = lax.axis_index("s") * info.num_cores + lax.axis_index("c")
        row_base = wid * rpw

        def e_sl(ch):
            return pl.ds((row_base + ch * _CH) * _DEG, ech)

        def out_sl(ch):
            return pl.ds(row_base + ch * _CH, _CH)

        def compute(p):
            for t in range(_CH // 2):
                vv = valbuf[p, pl.ds(16 * t, 16)]
                acc0 = [jnp.zeros((16,), jnp.float32)] * nj
                acc1 = [jnp.zeros((16,), jnp.float32)] * nj
                for d in range(_DEG):
                    w0 = _bcast_lane(vv, d)
                    w1 = _bcast_lane(vv, _DEG + d)
                    for j in range(nj):
                        acc0[j] = acc0[j] + w0 * gbuf[p, 16 * t + d, pl.ds(16 * j, 16)]
                        acc1[j] = acc1[j] + w1 * gbuf[p, 16 * t + _DEG + d, pl.ds(16 * j, 16)]
                for j in range(nj):
                    accbuf[p, 2 * t, pl.ds(16 * j, 16)] = acc0[j]
                    accbuf[p, 2 * t + 1, pl.ds(16 * j, 16)] = acc1[j]

        for k in range(nbuf):
            pltpu.async_copy(cols_hbm.at[e_sl(k)], colbuf.at[k], csem[k])
            pltpu.async_copy(vals_hbm.at[e_sl(k)], valbuf.at[k], vsem[k])
        for k in range(nbuf - 1):
            pltpu.make_async_copy(cols_hbm.at[e_sl(k)], colbuf.at[k], csem[k]).wait()
            pltpu.async_copy(xt_hbm.at[colbuf.at[k]], gbuf.at[k], gsem[k])

        def body(chb, carry):
            for b in range(nbuf):
                ch = chb * nbuf + b
                p = b
                q = (b - 1) % nbuf
                pltpu.make_async_copy(
                    xt_hbm.at[colbuf.at[p]], gbuf.at[p], gsem[p]).wait()

                @pl.when(ch + nbuf - 1 < nch)
                def _():
                    pltpu.make_async_copy(
                        cols_hbm.at[e_sl(ch + nbuf - 1)], colbuf.at[q], csem[q]).wait()
                    pltpu.async_copy(xt_hbm.at[colbuf.at[q]], gbuf.at[q], gsem[q])

                @pl.when(ch + nbuf < nch)
                def _():
                    pltpu.async_copy(cols_hbm.at[e_sl(ch + nbuf)], colbuf.at[p], csem[p])

                pltpu.make_async_copy(
                    vals_hbm.at[e_sl(ch)], valbuf.at[p], vsem[p]).wait()

                @pl.when(ch >= nbuf)
                def _():
                    pltpu.make_async_copy(
                        accbuf.at[p], out_hbm.at[out_sl(ch - nbuf)], osem[p]).wait()

                compute(p)
                pltpu.async_copy(accbuf.at[p], out_hbm.at[out_sl(ch)], osem[p])

                @pl.when(ch + nbuf < nch)
                def _():
                    pltpu.async_copy(vals_hbm.at[e_sl(ch + nbuf)], valbuf.at[p], vsem[p])
            return carry

        lax.fori_loop(0, nch // nbuf, body, 0)
        for k in range(nbuf):
            pltpu.make_async_copy(
                accbuf.at[k], out_hbm.at[out_sl(nch - nbuf + k)], osem[k]).wait()

    return spmv


def _spmv(xt, cols, vals, V, C):
    return _make_spmv(V, C)(xt, cols, vals)


def _mm3(x0, x1, m, wa, wb, wc, V, C, C2):
    ng = V // _VB

    def body(x0_ref, x1_ref, m_ref, a_ref, b_ref, c_ref, y_ref, st_ref):
        dot = functools.partial(jnp.dot, preferred_element_type=jnp.float32)
        x2 = 2.0 * m_ref[...] - x0_ref[...]
        y = dot(x0_ref[...], a_ref[...])
        y = y + dot(x1_ref[...], b_ref[...])
        y = y + dot(x2, c_ref[...])
        y_ref[...] = y

        @pl.when(pl.program_id(0) == 0)
        def _():
            st_ref[...] = jnp.zeros_like(st_ref)

        st_ref[0:1, :] += jnp.sum(y, axis=0, keepdims=True)
        st_ref[1:2, :] += jnp.sum(y * y, axis=0, keepdims=True)

    xspec = pl.BlockSpec((_VB, C), lambda i: (i, 0))
    wspec = pl.BlockSpec((C, C2), lambda i: (0, 0))
    return pl.pallas_call(
        body,
        grid=(ng,),
        in_specs=[xspec, xspec, xspec, wspec, wspec, wspec],
        out_specs=[pl.BlockSpec((_VB, C2), lambda i: (i, 0)),
                   pl.BlockSpec((8, C2), lambda i: (0, 0))],
        out_shape=[jax.ShapeDtypeStruct((V, C2), jnp.float32),
                   jax.ShapeDtypeStruct((8, C2), jnp.float32)],
    )(x0, x1, m, wa, wb, wc)


def _bn(y, st, gamma, beta, V, C2, want_out, want_pool):
    F = C2 // 2
    ng = V // _VB
    n = 2.0 * V

    def body(y_ref, st_ref, g_ref, b_ref, *out_refs):
        s = st_ref[0:1, :]
        q = st_ref[1:2, :]
        mean = (s[:, :F] + s[:, F:]) / n
        var = (q[:, :F] + q[:, F:]) / n - mean * mean
        scale = g_ref[...] / jnp.sqrt(var + 1e-5)
        shift = b_ref[...] - mean * scale
        yb = y_ref[...]
        z0 = jnp.maximum(yb[:, :F] * scale + shift, 0.0)
        z1 = jnp.maximum(yb[:, F:] * scale + shift, 0.0)
        k = 0
        if want_out:
            out_refs[k][0, :, :] = z0
            out_refs[k][1, :, :] = z1
            k += 1
        if want_pool:
            p0 = jnp.max(z0.reshape(_VB // 4, 4, F), axis=1)
            p1 = jnp.max(z1.reshape(_VB // 4, 4, F), axis=1)
            out_refs[k][:, :F] = p0
            out_refs[k][:, F:] = p1
            k += 1
        if not (want_out or want_pool):
            out_refs[0][:, :F] = z0
            out_refs[0][:, F:] = z1

    out_specs, out_shape = [], []
    if want_out:
        out_specs.append(pl.BlockSpec((2, _VB, F), lambda i: (0, i, 0)))
        out_shape.append(jax.ShapeDtypeStruct((2, V, F), jnp.float32))
    if want_pool:
        out_specs.append(pl.BlockSpec((_VB // 4, C2), lambda i: (i, 0)))
        out_shape.append(jax.ShapeDtypeStruct((V // 4, C2), jnp.float32))
    if not (want_out or want_pool):
        out_specs.append(pl.BlockSpec((_VB, C2), lambda i: (i, 0)))
        out_shape.append(jax.ShapeDtypeStruct((V, C2), jnp.float32))

    res = pl.pallas_call(
        body,
        grid=(ng,),
        in_specs=[pl.BlockSpec((_VB, C2), lambda i: (i, 0)),
                  pl.BlockSpec((8, C2), lambda i: (0, 0)),
                  pl.BlockSpec((1, F), lambda i: (0, 0)),
                  pl.BlockSpec((1, F), lambda i: (0, 0))],
        out_specs=out_specs,
        out_shape=out_shape,
    )(y, st, gamma.reshape(1, F), beta.reshape(1, F))
    return res if len(out_shape) > 1 else res[0]


def _blkdiag2(w):
    fi, fo = w.shape
    z = jnp.zeros((2 * fi, 2 * fo), jnp.float32)
    return z.at[:fi, :fo].set(w).at[fi:, fo:].set(w)


def _conv_bn(xt, cols, vals, W, gamma, beta, V, fin, fout, want_out, want_pool):
    C = 2 * fin
    C2 = 2 * fout
    w0, w1, w2 = W[0::3], W[1::3], W[2::3]
    x1 = _spmv(xt, cols, vals, V, C)
    m = _spmv(x1, cols, vals, V, C)
    y, st = _mm3(xt, x1, m, _blkdiag2(w0), _blkdiag2(w1), _blkdiag2(w2), V, C, C2)
    return _bn(y, st, gamma, beta, V, C2, want_out, want_pool)


def kernel(x, rows0, cols0, vals0, rows1, cols1, vals1, rows2, cols2, vals2,
           W1a, g1a, b1a, W1b, g1b, b1b, W2, g2, b2, W3, g3, b3):
    B, V0, F0 = x.shape
    V1, V2 = V0 // 4, V0 // 16
    xt0 = jnp.transpose(x, (1, 0, 2)).reshape(V0, B * F0)
    h = _conv_bn(xt0, cols0, vals0, W1a, g1a, b1a, V0, 16, 32, False, False)
    out1, p1 = _conv_bn(h, cols0, vals0, W1b, g1b, b1b, V0, 32, 64, True, True)
    out2, p2 = _conv_bn(p1, cols1, vals1, W2, g2, b2, V1, 64, 128, True, True)
    out3 = _conv_bn(p2, cols2, vals2, W3, g3, b3, V2, 128, 256, True, False)
    return (out3, out2, out1)


# speedup vs baseline: 1.2796x; 1.1065x over previous
import functools

import jax
import jax.numpy as jnp
from jax import lax
from jax.experimental import pallas as pl
from jax.experimental.pallas import tpu as pltpu
from jax.experimental.pallas import tpu_sc as plsc

_DEG = 8
_CH = 16
_VB = 1024


def _bcast_lane(vec, lane):
    idx = jnp.full((16, 1), lane, dtype=jnp.int32)
    dn = lax.GatherDimensionNumbers(
        offset_dims=(), collapsed_slice_dims=(0,), start_index_map=(0,))
    return lax.gather(vec, idx, dn, (1,),
                      mode=lax.GatherScatterMode.PROMISE_IN_BOUNDS)


def _make_spmv(V, C):
    info = plsc.get_sparse_core_info()
    nw = info.num_cores * info.num_subcores
    rpw = V // nw
    nch = rpw // _CH
    ech = _CH * _DEG
    nj = C // 16
    nbuf = 2
    mesh = plsc.VectorSubcoreMesh(core_axis_name="c", subcore_axis_name="s")

    @functools.partial(
        pl.kernel, mesh=mesh,
        out_type=jax.ShapeDtypeStruct((V, C), jnp.float32),
        scratch_types=[
            pltpu.VMEM((nbuf, ech), jnp.int32),
            pltpu.VMEM((nbuf, ech), jnp.float32),
            pltpu.VMEM((nbuf, ech, C), jnp.float32),
            pltpu.VMEM((nbuf, _CH, C), jnp.float32),
        ] + [pltpu.SemaphoreType.DMA] * (4 * nbuf),
        compiler_params=pltpu.CompilerParams(use_tc_tiling_on_sc=False),
    )
    def spmv(xt_hbm, cols_hbm, vals_hbm, out_hbm, colbuf, valbuf, gbuf, accbuf, *sems):
        csem, vsem = sems[0:nbuf], sems[nbuf:2 * nbuf]
        gsem, osem = sems[2 * nbuf:3 * nbuf], sems[3 * nbuf:4 * nbuf]
        wid = lax.axis_index("s") * info.num_cores + lax.axis_index("c")
        row_base = wid * rpw

        def e_sl(ch):
            return pl.ds((row_base + ch * _CH) * _DEG, ech)

        def out_sl(ch):
            return pl.ds(row_base + ch * _CH, _CH)

        def compute(p):
            for t in range(_CH // 2):
                vv = valbuf[p, pl.ds(16 * t, 16)]
                acc0 = [jnp.zeros((16,), jnp.float32)] * nj
                acc1 = [jnp.zeros((16,), jnp.float32)] * nj
                for d in range(_DEG):
                    w0 = _bcast_lane(vv, d)
                    w1 = _bcast_lane(vv, _DEG + d)
                    for j in range(nj):
                        acc0[j] = acc0[j] + w0 * gbuf[p, 16 * t + d, pl.ds(16 * j, 16)]
                        acc1[j] = acc1[j] + w1 * gbuf[p, 16 * t + _DEG + d, pl.ds(16 * j, 16)]
                for j in range(nj):
                    accbuf[p, 2 * t, pl.ds(16 * j, 16)] = acc0[j]
                    accbuf[p, 2 * t + 1, pl.ds(16 * j, 16)] = acc1[j]

        for k in range(nbuf):
            pltpu.async_copy(cols_hbm.at[e_sl(k)], colbuf.at[k], csem[k])
            pltpu.async_copy(vals_hbm.at[e_sl(k)], valbuf.at[k], vsem[k])
        for k in range(nbuf - 1):
            pltpu.make_async_copy(cols_hbm.at[e_sl(k)], colbuf.at[k], csem[k]).wait()
            pltpu.async_copy(xt_hbm.at[colbuf.at[k]], gbuf.at[k], gsem[k])

        def body(chb, carry):
            for b in range(nbuf):
                ch = chb * nbuf + b
                p = b
                q = (b - 1) % nbuf
                pltpu.make_async_copy(
                    xt_hbm.at[colbuf.at[p]], gbuf.at[p], gsem[p]).wait()

                @pl.when(ch + nbuf - 1 < nch)
                def _():
                    pltpu.make_async_copy(
                        cols_hbm.at[e_sl(ch + nbuf - 1)], colbuf.at[q], csem[q]).wait()
                    pltpu.async_copy(xt_hbm.at[colbuf.at[q]], gbuf.at[q], gsem[q])

                @pl.when(ch + nbuf < nch)
                def _():
                    pltpu.async_copy(cols_hbm.at[e_sl(ch + nbuf)], colbuf.at[p], csem[p])

                pltpu.make_async_copy(
                    vals_hbm.at[e_sl(ch)], valbuf.at[p], vsem[p]).wait()

                @pl.when(ch >= nbuf)
                def _():
                    pltpu.make_async_copy(
                        accbuf.at[p], out_hbm.at[out_sl(ch - nbuf)], osem[p]).wait()

                compute(p)
                pltpu.async_copy(accbuf.at[p], out_hbm.at[out_sl(ch)], osem[p])

                @pl.when(ch + nbuf < nch)
                def _():
                    pltpu.async_copy(vals_hbm.at[e_sl(ch + nbuf)], valbuf.at[p], vsem[p])
            return carry

        lax.fori_loop(0, nch // nbuf, body, 0)
        for k in range(nbuf):
            pltpu.make_async_copy(
                accbuf.at[k], out_hbm.at[out_sl(nch - nbuf + k)], osem[k]).wait()

    return spmv


def _spmv(xt, cols, vals, V, C):
    return _make_spmv(V, C)(xt, cols, vals)


def _mm3(x0, x1, m, wa, wb, wc, V, C, C2):
    ng = V // _VB

    def body(x0_ref, x1_ref, m_ref, a_ref, b_ref, c_ref, y_ref, st_ref):
        dot = functools.partial(jnp.dot, preferred_element_type=jnp.float32)
        x2 = 2.0 * m_ref[...] - x0_ref[...]
        y = dot(x0_ref[...], a_ref[...])
        y = y + dot(x1_ref[...], b_ref[...])
        y = y + dot(x2, c_ref[...])
        y_ref[...] = y

        @pl.when(pl.program_id(0) == 0)
        def _():
            st_ref[...] = jnp.zeros_like(st_ref)

        st_ref[0:1, :] += jnp.sum(y, axis=0, keepdims=True)
        st_ref[1:2, :] += jnp.sum(y * y, axis=0, keepdims=True)

    xspec = pl.BlockSpec((_VB, C), lambda i: (i, 0))
    wspec = pl.BlockSpec((C, C2), lambda i: (0, 0))
    return pl.pallas_call(
        body,
        grid=(ng,),
        in_specs=[xspec, xspec, xspec, wspec, wspec, wspec],
        out_specs=[pl.BlockSpec((_VB, C2), lambda i: (i, 0)),
                   pl.BlockSpec((8, C2), lambda i: (0, 0))],
        out_shape=[jax.ShapeDtypeStruct((V, C2), jnp.float32),
                   jax.ShapeDtypeStruct((8, C2), jnp.float32)],
    )(x0, x1, m, wa, wb, wc)


def _bn(y, st, gamma, beta, V, C2, want_out, want_pool):
    F = C2 // 2
    ng = V // _VB
    n = 2.0 * V

    def body(y_ref, st_ref, g_ref, b_ref, *out_refs):
        s = st_ref[0:1, :]
        q = st_ref[1:2, :]
        mean = (s[:, :F] + s[:, F:]) / n
        var = (q[:, :F] + q[:, F:]) / n - mean * mean
        scale = g_ref[...] / jnp.sqrt(var + 1e-5)
        shift = b_ref[...] - mean * scale
        yb = y_ref[...]
        z0 = jnp.maximum(yb[:, :F] * scale + shift, 0.0)
        z1 = jnp.maximum(yb[:, F:] * scale + shift, 0.0)
        k = 0
        if want_out:
            out_refs[k][0, :, :] = z0
            out_refs[k][1, :, :] = z1
            k += 1
        if want_pool:
            p0 = jnp.max(z0.reshape(_VB // 4, 4, F), axis=1)
            p1 = jnp.max(z1.reshape(_VB // 4, 4, F), axis=1)
            out_refs[k][:, :F] = p0
            out_refs[k][:, F:] = p1
            k += 1
        if not (want_out or want_pool):
            out_refs[0][:, :F] = z0
            out_refs[0][:, F:] = z1

    out_specs, out_shape = [], []
    if want_out:
        out_specs.append(pl.BlockSpec((2, _VB, F), lambda i: (0, i, 0)))
        out_shape.append(jax.ShapeDtypeStruct((2, V, F), jnp.float32))
    if want_pool:
        out_specs.append(pl.BlockSpec((_VB // 4, C2), lambda i: (i, 0)))
        out_shape.append(jax.ShapeDtypeStruct((V // 4, C2), jnp.float32))
    if not (want_out or want_pool):
        out_specs.append(pl.BlockSpec((_VB, C2), lambda i: (i, 0)))
        out_shape.append(jax.ShapeDtypeStruct((V, C2), jnp.float32))

    res = pl.pallas_call(
        body,
        grid=(ng,),
        in_specs=[pl.BlockSpec((_VB, C2), lambda i: (i, 0)),
                  pl.BlockSpec((8, C2), lambda i: (0, 0)),
                  pl.BlockSpec((1, F), lambda i: (0, 0)),
                  pl.BlockSpec((1, F), lambda i: (0, 0))],
        out_specs=out_specs,
        out_shape=out_shape,
    )(y, st, gamma.reshape(1, F), beta.reshape(1, F))
    return res if len(out_shape) > 1 else res[0]


def _blkdiag2(w):
    fi, fo = w.shape
    z = jnp.zeros((2 * fi, 2 * fo), jnp.float32)
    return z.at[:fi, :fo].set(w).at[fi:, fo:].set(w)


def _conv_bn(xt, cols, vals, W, gamma, beta, V, fin, fout, want_out, want_pool):
    C = 2 * fin
    C2 = 2 * fout
    w0, w1, w2 = W[0::3], W[1::3], W[2::3]
    x1 = _spmv(xt, cols, vals, V, C)
    m = _spmv(x1, cols, vals, V, C)
    y, st = _mm3(xt, x1, m, _blkdiag2(w0), _blkdiag2(w1), _blkdiag2(w2), V, C, C2)
    return _bn(y, st, gamma, beta, V, C2, want_out, want_pool)


def kernel(x, rows0, cols0, vals0, rows1, cols1, vals1, rows2, cols2, vals2,
           W1a, g1a, b1a, W1b, g1b, b1b, W2, g2, b2, W3, g3, b3):
    B, V0, F0 = x.shape
    V1, V2 = V0 // 4, V0 // 16
    xt0 = jnp.transpose(x, (1, 0, 2)).reshape(V0, B * F0)
    h = _conv_bn(xt0, cols0, vals0, W1a, g1a, b1a, V0, 16, 32, False, False)
    out1, p1 = _conv_bn(h, cols0, vals0, W1b, g1b, b1b, V0, 32, 64, True, True)
    out2, p2 = _conv_bn(p1, cols1, vals1, W2, g2, b2, V1, 64, 128, True, True)
    out3 = _conv_bn(p2, cols2, vals2, W3, g3, b3, V2, 128, 256, True, False)
    return (out3, out2, out1)


# speedup vs baseline: 1.3551x; 1.0590x over previous
import functools

import jax
import jax.numpy as jnp
from jax import lax
from jax.experimental import pallas as pl
from jax.experimental.pallas import tpu as pltpu
from jax.experimental.pallas import tpu_sc as plsc

_DEG = 8
_CH = 16
def _vb(V):
    return 2048 if V % 2048 == 0 else 1024


def _bcast_lane(vec, lane):
    idx = jnp.full((16, 1), lane, dtype=jnp.int32)
    dn = lax.GatherDimensionNumbers(
        offset_dims=(), collapsed_slice_dims=(0,), start_index_map=(0,))
    return lax.gather(vec, idx, dn, (1,),
                      mode=lax.GatherScatterMode.PROMISE_IN_BOUNDS)


def _make_spmv(V, C):
    info = plsc.get_sparse_core_info()
    nw = info.num_cores * info.num_subcores
    rpw = V // nw
    nch = rpw // _CH
    ech = _CH * _DEG
    nj = C // 16
    nbuf = 2
    mesh = plsc.VectorSubcoreMesh(core_axis_name="c", subcore_axis_name="s")

    @functools.partial(
        pl.kernel, mesh=mesh,
        out_type=jax.ShapeDtypeStruct((V, C), jnp.float32),
        scratch_types=[
            pltpu.VMEM((nbuf, ech), jnp.int32),
            pltpu.VMEM((nbuf, ech), jnp.float32),
            pltpu.VMEM((nbuf, ech, C), jnp.float32),
            pltpu.VMEM((nbuf, _CH, C), jnp.float32),
        ] + [pltpu.SemaphoreType.DMA] * (4 * nbuf),
        compiler_params=pltpu.CompilerParams(use_tc_tiling_on_sc=False),
    )
    def spmv(xt_hbm, cols_hbm, vals_hbm, out_hbm, colbuf, valbuf, gbuf, accbuf, *sems):
        csem, vsem = sems[0:nbuf], sems[nbuf:2 * nbuf]
        gsem, osem = sems[2 * nbuf:3 * nbuf], sems[3 * nbuf:4 * nbuf]
        wid = lax.axis_index("s") * info.num_cores + lax.axis_index("c")
        row_base = wid * rpw

        def e_sl(ch):
            return pl.ds((row_base + ch * _CH) * _DEG, ech)

        def out_sl(ch):
            return pl.ds(row_base + ch * _CH, _CH)

        def compute(p):
            for t in range(_CH // 2):
                vv = valbuf[p, pl.ds(16 * t, 16)]
                acc0 = [jnp.zeros((16,), jnp.float32)] * nj
                acc1 = [jnp.zeros((16,), jnp.float32)] * nj
                for d in range(_DEG):
                    w0 = _bcast_lane(vv, d)
                    w1 = _bcast_lane(vv, _DEG + d)
                    for j in range(nj):
                        acc0[j] = acc0[j] + w0 * gbuf[p, 16 * t + d, pl.ds(16 * j, 16)]
                        acc1[j] = acc1[j] + w1 * gbuf[p, 16 * t + _DEG + d, pl.ds(16 * j, 16)]
                for j in range(nj):
                    accbuf[p, 2 * t, pl.ds(16 * j, 16)] = acc0[j]
                    accbuf[p, 2 * t + 1, pl.ds(16 * j, 16)] = acc1[j]

        for k in range(nbuf):
            pltpu.async_copy(cols_hbm.at[e_sl(k)], colbuf.at[k], csem[k])
            pltpu.async_copy(vals_hbm.at[e_sl(k)], valbuf.at[k], vsem[k])
        for k in range(nbuf - 1):
            pltpu.make_async_copy(cols_hbm.at[e_sl(k)], colbuf.at[k], csem[k]).wait()
            pltpu.async_copy(xt_hbm.at[colbuf.at[k]], gbuf.at[k], gsem[k])

        def body(chb, carry):
            for b in range(nbuf):
                ch = chb * nbuf + b
                p = b
                q = (b - 1) % nbuf
                pltpu.make_async_copy(
                    xt_hbm.at[colbuf.at[p]], gbuf.at[p], gsem[p]).wait()

                @pl.when(ch + nbuf - 1 < nch)
                def _():
                    pltpu.make_async_copy(
                        cols_hbm.at[e_sl(ch + nbuf - 1)], colbuf.at[q], csem[q]).wait()
                    pltpu.async_copy(xt_hbm.at[colbuf.at[q]], gbuf.at[q], gsem[q])

                @pl.when(ch + nbuf < nch)
                def _():
                    pltpu.async_copy(cols_hbm.at[e_sl(ch + nbuf)], colbuf.at[p], csem[p])

                pltpu.make_async_copy(
                    vals_hbm.at[e_sl(ch)], valbuf.at[p], vsem[p]).wait()

                @pl.when(ch >= nbuf)
                def _():
                    pltpu.make_async_copy(
                        accbuf.at[p], out_hbm.at[out_sl(ch - nbuf)], osem[p]).wait()

                compute(p)
                pltpu.async_copy(accbuf.at[p], out_hbm.at[out_sl(ch)], osem[p])

                @pl.when(ch + nbuf < nch)
                def _():
                    pltpu.async_copy(vals_hbm.at[e_sl(ch + nbuf)], valbuf.at[p], vsem[p])
            return carry

        lax.fori_loop(0, nch // nbuf, body, 0)
        for k in range(nbuf):
            pltpu.make_async_copy(
                accbuf.at[k], out_hbm.at[out_sl(nch - nbuf + k)], osem[k]).wait()

    return spmv


def _spmv(xt, cols, vals, V, C):
    return _make_spmv(V, C)(xt, cols, vals)


def _mm3(x0, x1, m, wa, wb, wc, V, C, C2):
    _VB = _vb(V)
    ng = V // _VB

    def body(x0_ref, x1_ref, m_ref, a_ref, b_ref, c_ref, y_ref, st_ref):
        dot = functools.partial(jnp.dot, preferred_element_type=jnp.float32)
        x2 = 2.0 * m_ref[...] - x0_ref[...]
        y = dot(x0_ref[...], a_ref[...])
        y = y + dot(x1_ref[...], b_ref[...])
        y = y + dot(x2, c_ref[...])
        y_ref[...] = y

        @pl.when(pl.program_id(0) == 0)
        def _():
            st_ref[...] = jnp.zeros_like(st_ref)

        st_ref[0:1, :] += jnp.sum(y, axis=0, keepdims=True)
        st_ref[1:2, :] += jnp.sum(y * y, axis=0, keepdims=True)

    xspec = pl.BlockSpec((_VB, C), lambda i: (i, 0))
    wspec = pl.BlockSpec((C, C2), lambda i: (0, 0))
    return pl.pallas_call(
        body,
        grid=(ng,),
        in_specs=[xspec, xspec, xspec, wspec, wspec, wspec],
        out_specs=[pl.BlockSpec((_VB, C2), lambda i: (i, 0)),
                   pl.BlockSpec((8, C2), lambda i: (0, 0))],
        out_shape=[jax.ShapeDtypeStruct((V, C2), jnp.float32),
                   jax.ShapeDtypeStruct((8, C2), jnp.float32)],
    )(x0, x1, m, wa, wb, wc)


def _bn(y, st, gamma, beta, V, C2, want_out, want_pool):
    F = C2 // 2
    _VB = _vb(V)
    ng = V // _VB
    n = 2.0 * V

    def body(y_ref, st_ref, g_ref, b_ref, *out_refs):
        s = st_ref[0:1, :]
        q = st_ref[1:2, :]
        mean = (s[:, :F] + s[:, F:]) / n
        var = (q[:, :F] + q[:, F:]) / n - mean * mean
        scale = g_ref[...] / jnp.sqrt(var + 1e-5)
        shift = b_ref[...] - mean * scale
        yb = y_ref[...]
        z0 = jnp.maximum(yb[:, :F] * scale + shift, 0.0)
        z1 = jnp.maximum(yb[:, F:] * scale + shift, 0.0)
        k = 0
        if want_out:
            out_refs[k][0, :, :] = z0
            out_refs[k][1, :, :] = z1
            k += 1
        if want_pool:
            p0 = jnp.max(z0.reshape(_VB // 4, 4, F), axis=1)
            p1 = jnp.max(z1.reshape(_VB // 4, 4, F), axis=1)
            out_refs[k][:, :F] = p0
            out_refs[k][:, F:] = p1
            k += 1
        if not (want_out or want_pool):
            out_refs[0][:, :F] = z0
            out_refs[0][:, F:] = z1

    out_specs, out_shape = [], []
    if want_out:
        out_specs.append(pl.BlockSpec((2, _VB, F), lambda i: (0, i, 0)))
        out_shape.append(jax.ShapeDtypeStruct((2, V, F), jnp.float32))
    if want_pool:
        out_specs.append(pl.BlockSpec((_VB // 4, C2), lambda i: (i, 0)))
        out_shape.append(jax.ShapeDtypeStruct((V // 4, C2), jnp.float32))
    if not (want_out or want_pool):
        out_specs.append(pl.BlockSpec((_VB, C2), lambda i: (i, 0)))
        out_shape.append(jax.ShapeDtypeStruct((V, C2), jnp.float32))

    res = pl.pallas_call(
        body,
        grid=(ng,),
        in_specs=[pl.BlockSpec((_VB, C2), lambda i: (i, 0)),
                  pl.BlockSpec((8, C2), lambda i: (0, 0)),
                  pl.BlockSpec((1, F), lambda i: (0, 0)),
                  pl.BlockSpec((1, F), lambda i: (0, 0))],
        out_specs=out_specs,
        out_shape=out_shape,
    )(y, st, gamma.reshape(1, F), beta.reshape(1, F))
    return res if len(out_shape) > 1 else res[0]


def _blkdiag2(w):
    fi, fo = w.shape
    z = jnp.zeros((2 * fi, 2 * fo), jnp.float32)
    return z.at[:fi, :fo].set(w).at[fi:, fo:].set(w)


def _conv_bn(xt, cols, vals, W, gamma, beta, V, fin, fout, want_out, want_pool):
    C = 2 * fin
    C2 = 2 * fout
    w0, w1, w2 = W[0::3], W[1::3], W[2::3]
    x1 = _spmv(xt, cols, vals, V, C)
    m = _spmv(x1, cols, vals, V, C)
    y, st = _mm3(xt, x1, m, _blkdiag2(w0), _blkdiag2(w1), _blkdiag2(w2), V, C, C2)
    return _bn(y, st, gamma, beta, V, C2, want_out, want_pool)


def kernel(x, rows0, cols0, vals0, rows1, cols1, vals1, rows2, cols2, vals2,
           W1a, g1a, b1a, W1b, g1b, b1b, W2, g2, b2, W3, g3, b3):
    B, V0, F0 = x.shape
    V1, V2 = V0 // 4, V0 // 16
    xt0 = jnp.transpose(x, (1, 0, 2)).reshape(V0, B * F0)
    h = _conv_bn(xt0, cols0, vals0, W1a, g1a, b1a, V0, 16, 32, False, False)
    out1, p1 = _conv_bn(h, cols0, vals0, W1b, g1b, b1b, V0, 32, 64, True, True)
    out2, p2 = _conv_bn(p1, cols1, vals1, W2, g2, b2, V1, 64, 128, True, True)
    out3 = _conv_bn(p2, cols2, vals2, W3, g3, b3, V2, 128, 256, True, False)
    return (out3, out2, out1)


# speedup vs baseline: 1.4018x; 1.0345x over previous
import functools

import jax
import jax.numpy as jnp
from jax import lax
from jax.experimental import pallas as pl
from jax.experimental.pallas import tpu as pltpu
from jax.experimental.pallas import tpu_sc as plsc

_DEG = 8
_CH = 16
def _vb(V):
    return 2048 if V % 2048 == 0 else 1024


def _bcast_lane(vec, lane):
    idx = jnp.full((16, 1), lane, dtype=jnp.int32)
    dn = lax.GatherDimensionNumbers(
        offset_dims=(), collapsed_slice_dims=(0,), start_index_map=(0,))
    return lax.gather(vec, idx, dn, (1,),
                      mode=lax.GatherScatterMode.PROMISE_IN_BOUNDS)


def _make_spmv(V, C):
    info = plsc.get_sparse_core_info()
    nw = info.num_cores * info.num_subcores
    rpw = V // nw
    ch_rows = 32 if C <= 64 else 16
    nch = rpw // ch_rows
    ech = ch_rows * _DEG
    nseg = ech // 128
    nj = C // 16
    nbuf = 2
    mesh = plsc.VectorSubcoreMesh(core_axis_name="c", subcore_axis_name="s")

    @functools.partial(
        pl.kernel, mesh=mesh,
        out_type=jax.ShapeDtypeStruct((V, C), jnp.float32),
        scratch_types=[
            pltpu.VMEM((nbuf, nseg, 128), jnp.int32),
            pltpu.VMEM((nbuf, nseg, 128), jnp.float32),
            pltpu.VMEM((nbuf, ech, C), jnp.float32),
            pltpu.VMEM((nbuf, ch_rows, C), jnp.float32),
        ] + [pltpu.SemaphoreType.DMA] * (4 * nbuf),
        compiler_params=pltpu.CompilerParams(use_tc_tiling_on_sc=False),
    )
    def spmv(xt_hbm, cols_hbm, vals_hbm, out_hbm, colbuf, valbuf, gbuf, accbuf, *sems):
        csem, vsem = sems[0:nbuf], sems[nbuf:2 * nbuf]
        gsem, osem = sems[2 * nbuf:3 * nbuf], sems[3 * nbuf:4 * nbuf]
        wid = lax.axis_index("s") * info.num_cores + lax.axis_index("c")
        row_base = wid * rpw

        def e_sl(ch):
            return pl.ds((row_base + ch * ch_rows) // 16, nseg)

        def out_sl(ch):
            return pl.ds(row_base + ch * ch_rows, ch_rows)

        def start_gathers(p):
            for sgi in range(nseg):
                pltpu.async_copy(xt_hbm.at[colbuf.at[p, sgi]],
                                 gbuf.at[p, pl.ds(128 * sgi, 128)], gsem[p])

        def wait_gathers(p):
            for sgi in range(nseg):
                pltpu.make_async_copy(xt_hbm.at[colbuf.at[p, sgi]],
                                      gbuf.at[p, pl.ds(128 * sgi, 128)], gsem[p]).wait()

        def compute(p):
            for t in range(ch_rows // 2):
                vv = valbuf[p, t // 8, pl.ds(16 * (t % 8), 16)]
                acc0 = [jnp.zeros((16,), jnp.float32)] * nj
                acc1 = [jnp.zeros((16,), jnp.float32)] * nj
                for d in range(_DEG):
                    w0 = _bcast_lane(vv, d)
                    w1 = _bcast_lane(vv, _DEG + d)
                    for j in range(nj):
                        acc0[j] = acc0[j] + w0 * gbuf[p, 16 * t + d, pl.ds(16 * j, 16)]
                        acc1[j] = acc1[j] + w1 * gbuf[p, 16 * t + _DEG + d, pl.ds(16 * j, 16)]
                for j in range(nj):
                    accbuf[p, 2 * t, pl.ds(16 * j, 16)] = acc0[j]
                    accbuf[p, 2 * t + 1, pl.ds(16 * j, 16)] = acc1[j]

        for k in range(nbuf):
            pltpu.async_copy(cols_hbm.at[e_sl(k)], colbuf.at[k], csem[k])
            pltpu.async_copy(vals_hbm.at[e_sl(k)], valbuf.at[k], vsem[k])
        for k in range(nbuf - 1):
            pltpu.make_async_copy(cols_hbm.at[e_sl(k)], colbuf.at[k], csem[k]).wait()
            start_gathers(k)

        def body(chb, carry):
            for b in range(nbuf):
                ch = chb * nbuf + b
                p = b
                q = (b - 1) % nbuf
                wait_gathers(p)

                @pl.when(ch + nbuf - 1 < nch)
                def _():
                    pltpu.make_async_copy(
                        cols_hbm.at[e_sl(ch + nbuf - 1)], colbuf.at[q], csem[q]).wait()
                    start_gathers(q)

                @pl.when(ch + nbuf < nch)
                def _():
                    pltpu.async_copy(cols_hbm.at[e_sl(ch + nbuf)], colbuf.at[p], csem[p])

                pltpu.make_async_copy(
                    vals_hbm.at[e_sl(ch)], valbuf.at[p], vsem[p]).wait()

                @pl.when(ch >= nbuf)
                def _():
                    pltpu.make_async_copy(
                        accbuf.at[p], out_hbm.at[out_sl(ch - nbuf)], osem[p]).wait()

                compute(p)
                pltpu.async_copy(accbuf.at[p], out_hbm.at[out_sl(ch)], osem[p])

                @pl.when(ch + nbuf < nch)
                def _():
                    pltpu.async_copy(vals_hbm.at[e_sl(ch + nbuf)], valbuf.at[p], vsem[p])
            return carry

        lax.fori_loop(0, nch // nbuf, body, 0)
        for k in range(nbuf):
            pltpu.make_async_copy(
                accbuf.at[k], out_hbm.at[out_sl(nch - nbuf + k)], osem[k]).wait()

    return spmv


def _spmv(xt, cols, vals, V, C):
    return _make_spmv(V, C)(xt, cols.reshape(-1, 128), vals.reshape(-1, 128))


def _mm3(x0, x1, m, wa, wb, wc, V, C, C2):
    _VB = _vb(V)
    ng = V // _VB

    def body(x0_ref, x1_ref, m_ref, a_ref, b_ref, c_ref, y_ref, st_ref):
        dot = functools.partial(jnp.dot, preferred_element_type=jnp.float32)
        x2 = 2.0 * m_ref[...] - x0_ref[...]
        y = dot(x0_ref[...], a_ref[...])
        y = y + dot(x1_ref[...], b_ref[...])
        y = y + dot(x2, c_ref[...])
        y_ref[...] = y

        @pl.when(pl.program_id(0) == 0)
        def _():
            st_ref[...] = jnp.zeros_like(st_ref)

        st_ref[0:1, :] += jnp.sum(y, axis=0, keepdims=True)
        st_ref[1:2, :] += jnp.sum(y * y, axis=0, keepdims=True)

    xspec = pl.BlockSpec((_VB, C), lambda i: (i, 0))
    wspec = pl.BlockSpec((C, C2), lambda i: (0, 0))
    return pl.pallas_call(
        body,
        grid=(ng,),
        in_specs=[xspec, xspec, xspec, wspec, wspec, wspec],
        out_specs=[pl.BlockSpec((_VB, C2), lambda i: (i, 0)),
                   pl.BlockSpec((8, C2), lambda i: (0, 0))],
        out_shape=[jax.ShapeDtypeStruct((V, C2), jnp.float32),
                   jax.ShapeDtypeStruct((8, C2), jnp.float32)],
    )(x0, x1, m, wa, wb, wc)


def _bn(y, st, gamma, beta, V, C2, want_out, want_pool):
    F = C2 // 2
    _VB = _vb(V)
    ng = V // _VB
    n = 2.0 * V

    def body(y_ref, st_ref, g_ref, b_ref, *out_refs):
        s = st_ref[0:1, :]
        q = st_ref[1:2, :]
        mean = (s[:, :F] + s[:, F:]) / n
        var = (q[:, :F] + q[:, F:]) / n - mean * mean
        scale = g_ref[...] / jnp.sqrt(var + 1e-5)
        shift = b_ref[...] - mean * scale
        yb = y_ref[...]
        z0 = jnp.maximum(yb[:, :F] * scale + shift, 0.0)
        z1 = jnp.maximum(yb[:, F:] * scale + shift, 0.0)
        k = 0
        if want_out:
            out_refs[k][0, :, :] = z0
            out_refs[k][1, :, :] = z1
            k += 1
        if want_pool:
            p0 = jnp.max(z0.reshape(_VB // 4, 4, F), axis=1)
            p1 = jnp.max(z1.reshape(_VB // 4, 4, F), axis=1)
            out_refs[k][:, :F] = p0
            out_refs[k][:, F:] = p1
            k += 1
        if not (want_out or want_pool):
            out_refs[0][:, :F] = z0
            out_refs[0][:, F:] = z1

    out_specs, out_shape = [], []
    if want_out:
        out_specs.append(pl.BlockSpec((2, _VB, F), lambda i: (0, i, 0)))
        out_shape.append(jax.ShapeDtypeStruct((2, V, F), jnp.float32))
    if want_pool:
        out_specs.append(pl.BlockSpec((_VB // 4, C2), lambda i: (i, 0)))
        out_shape.append(jax.ShapeDtypeStruct((V // 4, C2), jnp.float32))
    if not (want_out or want_pool):
        out_specs.append(pl.BlockSpec((_VB, C2), lambda i: (i, 0)))
        out_shape.append(jax.ShapeDtypeStruct((V, C2), jnp.float32))

    res = pl.pallas_call(
        body,
        grid=(ng,),
        in_specs=[pl.BlockSpec((_VB, C2), lambda i: (i, 0)),
                  pl.BlockSpec((8, C2), lambda i: (0, 0)),
                  pl.BlockSpec((1, F), lambda i: (0, 0)),
                  pl.BlockSpec((1, F), lambda i: (0, 0))],
        out_specs=out_specs,
        out_shape=out_shape,
    )(y, st, gamma.reshape(1, F), beta.reshape(1, F))
    return res if len(out_shape) > 1 else res[0]


def _blkdiag2(w):
    fi, fo = w.shape
    z = jnp.zeros((2 * fi, 2 * fo), jnp.float32)
    return z.at[:fi, :fo].set(w).at[fi:, fo:].set(w)


def _conv_bn(xt, cols, vals, W, gamma, beta, V, fin, fout, want_out, want_pool):
    C = 2 * fin
    C2 = 2 * fout
    w0, w1, w2 = W[0::3], W[1::3], W[2::3]
    x1 = _spmv(xt, cols, vals, V, C)
    m = _spmv(x1, cols, vals, V, C)
    y, st = _mm3(xt, x1, m, _blkdiag2(w0), _blkdiag2(w1), _blkdiag2(w2), V, C, C2)
    return _bn(y, st, gamma, beta, V, C2, want_out, want_pool)


def kernel(x, rows0, cols0, vals0, rows1, cols1, vals1, rows2, cols2, vals2,
           W1a, g1a, b1a, W1b, g1b, b1b, W2, g2, b2, W3, g3, b3):
    B, V0, F0 = x.shape
    V1, V2 = V0 // 4, V0 // 16
    xt0 = jnp.transpose(x, (1, 0, 2)).reshape(V0, B * F0)
    h = _conv_bn(xt0, cols0, vals0, W1a, g1a, b1a, V0, 16, 32, False, False)
    out1, p1 = _conv_bn(h, cols0, vals0, W1b, g1b, b1b, V0, 32, 64, True, True)
    out2, p2 = _conv_bn(p1, cols1, vals1, W2, g2, b2, V1, 64, 128, True, True)
    out3 = _conv_bn(p2, cols2, vals2, W3, g3, b3, V2, 128, 256, True, False)
    return (out3, out2, out1)
